# baseline (speedup 1.0000x reference)
import functools

import jax
import jax.numpy as jnp
from jax import lax
from jax.experimental import pallas as pl
from jax.experimental.pallas import tpu as pltpu
from jax.experimental.pallas import tpu_sc as plsc

N = 16384 * 100
NC, NS = 2, 16
NW = NC * NS
PER_W = N // NW
CHUNK = 12800
NCHUNK = PER_W // CHUNK
VECS = CHUNK // 16

_mesh = plsc.VectorSubcoreMesh(
    core_axis_name="c", subcore_axis_name="s", num_cores=NC, num_subcores=NS
)


@functools.partial(
    pl.kernel,
    out_type=jax.ShapeDtypeStruct((N,), jnp.float32),
    mesh=_mesh,
    scratch_types=[
        pltpu.VMEM((16,), jnp.float32),
        pltpu.VMEM((CHUNK,), jnp.int32),
        pltpu.VMEM((CHUNK,), jnp.int32),
        pltpu.VMEM((CHUNK,), jnp.float32),
    ],
)
def _sc_lookup(in_hbm, tg_hbm, tab_hbm, out_hbm, tab_v, in_v, tg_v, out_v):
    wid = lax.axis_index("s") * NC + lax.axis_index("c")
    base = wid * PER_W
    pltpu.sync_copy(tab_hbm, tab_v)
    tab = tab_v[...]

    def chunk_body(ci, _):
        off = base + ci * CHUNK
        pltpu.sync_copy(in_hbm.at[pl.ds(off, CHUNK)], in_v)
        pltpu.sync_copy(tg_hbm.at[pl.ds(off, CHUNK)], tg_v)

        def vec_body(vi, _):
            s = pl.ds(vi * 16, 16)
            idx = in_v[s] * 4 + tg_v[s]
            out_v[s] = tab.at[idx].get(mode="promise_in_bounds")
            return 0

        lax.fori_loop(0, VECS, vec_body, 0)
        pltpu.sync_copy(out_v, out_hbm.at[pl.ds(off, CHUNK)])
        return 0

    lax.fori_loop(0, NCHUNK, chunk_body, 0)


def kernel(input_, target, sim_lookup):
    out = _sc_lookup(
        input_.reshape(N).astype(jnp.int32),
        target.reshape(N).astype(jnp.int32),
        sim_lookup.astype(jnp.float32),
    )
    return out.reshape(input_.shape)


# speedup vs baseline: 1.1515x; 1.1515x over previous
import functools

import jax
import jax.numpy as jnp
from jax import lax
from jax.experimental import pallas as pl
from jax.experimental.pallas import tpu as pltpu
from jax.experimental.pallas import tpu_sc as plsc

N = 16384 * 100
NC, NS = 2, 16
NW = NC * NS
PER_W = N // NW
NBUF = 2
CHUNK = 12800
NCHUNK = PER_W // CHUNK
VECS = CHUNK // 16

_mesh = plsc.VectorSubcoreMesh(
    core_axis_name="c", subcore_axis_name="s", num_cores=NC, num_subcores=NS
)


@functools.partial(
    pl.kernel,
    out_type=jax.ShapeDtypeStruct((N,), jnp.float32),
    mesh=_mesh,
    scratch_types=[
        pltpu.VMEM((16,), jnp.float32),
        pltpu.VMEM((CHUNK,), jnp.int32), pltpu.VMEM((CHUNK,), jnp.int32),
        pltpu.VMEM((CHUNK,), jnp.int32), pltpu.VMEM((CHUNK,), jnp.int32),
        pltpu.VMEM((CHUNK,), jnp.float32), pltpu.VMEM((CHUNK,), jnp.float32),
        pltpu.SemaphoreType.DMA, pltpu.SemaphoreType.DMA,
        pltpu.SemaphoreType.DMA, pltpu.SemaphoreType.DMA,
    ],
)
def _sc_lookup(in_hbm, tg_hbm, tab_hbm, out_hbm,
               tab_v, in0, in1, tg0, tg1, out0, out1,
               si0, si1, so0, so1):
    wid = lax.axis_index("s") * NC + lax.axis_index("c")
    base = wid * PER_W
    pltpu.sync_copy(tab_hbm, tab_v)
    tab = tab_v[...]

    bufs = ((in0, tg0, out0, si0, so0), (in1, tg1, out1, si1, so1))

    for b in range(NBUF):
        off = base + b * CHUNK
        in_v, tg_v, _, sem_i, _ = bufs[b]
        pltpu.async_copy(in_hbm.at[pl.ds(off, CHUNK)], in_v, sem_i)
        pltpu.async_copy(tg_hbm.at[pl.ds(off, CHUNK)], tg_v, sem_i)

    for ci in range(NCHUNK):
        in_v, tg_v, out_v, sem_i, sem_o = bufs[ci % NBUF]
        off = base + ci * CHUNK
        pltpu.make_async_copy(in_hbm.at[pl.ds(off, CHUNK)], in_v, sem_i).wait()
        pltpu.make_async_copy(tg_hbm.at[pl.ds(off, CHUNK)], tg_v, sem_i).wait()
        if ci >= NBUF:
            prev = base + (ci - NBUF) * CHUNK
            pltpu.make_async_copy(
                out_v, out_hbm.at[pl.ds(prev, CHUNK)], sem_o).wait()

        @plsc.parallel_loop(0, VECS, 1, unroll=8)
        def _vec(vi):
            s = pl.ds(vi * 16, 16)
            idx = in_v[s] * 4 + tg_v[s]
            out_v[s] = tab.at[idx].get(mode="promise_in_bounds")

        pltpu.async_copy(out_v, out_hbm.at[pl.ds(off, CHUNK)], sem_o)
        if ci + NBUF < NCHUNK:
            noff = base + (ci + NBUF) * CHUNK
            pltpu.async_copy(in_hbm.at[pl.ds(noff, CHUNK)], in_v, sem_i)
            pltpu.async_copy(tg_hbm.at[pl.ds(noff, CHUNK)], tg_v, sem_i)

    for ci in range(NCHUNK - NBUF, NCHUNK):
        _, _, out_v, _, sem_o = bufs[ci % NBUF]
        off = base + ci * CHUNK
        pltpu.make_async_copy(out_v, out_hbm.at[pl.ds(off, CHUNK)], sem_o).wait()


def kernel(input_, target, sim_lookup):
    out = _sc_lookup(
        input_.reshape(N).astype(jnp.int32),
        target.reshape(N).astype(jnp.int32),
        sim_lookup.astype(jnp.float32),
    )
    return out.reshape(input_.shape)


# speedup vs baseline: 1.9149x; 1.6630x over previous
import functools

import jax
import jax.numpy as jnp
from jax import lax
from jax.experimental import pallas as pl
from jax.experimental.pallas import tpu as pltpu
from jax.experimental.pallas import tpu_sc as plsc

R, C = 16384, 100
NC, NS = 2, 16
NW = NC * NS
ROWS_W = R // NW
NBUF = 2
ROWS_C = 128
NCHUNK = ROWS_W // ROWS_C
WINS = (0, 16, 32, 48, 64, 80, 84)

_mesh = plsc.VectorSubcoreMesh(
    core_axis_name="c", subcore_axis_name="s", num_cores=NC, num_subcores=NS
)


@functools.partial(
    pl.kernel,
    out_type=jax.ShapeDtypeStruct((R, C), jnp.float32),
    mesh=_mesh,
    compiler_params=pltpu.CompilerParams(use_tc_tiling_on_sc=True),
    scratch_types=[
        pltpu.VMEM((16,), jnp.float32),
        pltpu.VMEM((ROWS_C, C), jnp.int32), pltpu.VMEM((ROWS_C, C), jnp.int32),
        pltpu.VMEM((ROWS_C, C), jnp.int32), pltpu.VMEM((ROWS_C, C), jnp.int32),
        pltpu.VMEM((ROWS_C, C), jnp.float32), pltpu.VMEM((ROWS_C, C), jnp.float32),
        pltpu.SemaphoreType.DMA, pltpu.SemaphoreType.DMA,
        pltpu.SemaphoreType.DMA, pltpu.SemaphoreType.DMA,
    ],
)
def _sc_lookup(in_hbm, tg_hbm, tab_hbm, out_hbm,
               tab_v, in0, in1, tg0, tg1, out0, out1,
               si0, si1, so0, so1):
    wid = lax.axis_index("s") * NC + lax.axis_index("c")
    base = wid * ROWS_W
    pltpu.sync_copy(tab_hbm, tab_v)
    tab = tab_v[...]

    bufs = ((in0, tg0, out0, si0, so0), (in1, tg1, out1, si1, so1))

    for b in range(NBUF):
        off = base + b * ROWS_C
        in_v, tg_v, _, sem_i, _ = bufs[b]
        pltpu.async_copy(in_hbm.at[pl.ds(off, ROWS_C), :], in_v, sem_i)
        pltpu.async_copy(tg_hbm.at[pl.ds(off, ROWS_C), :], tg_v, sem_i)

    for ci in range(NCHUNK):
        in_v, tg_v, out_v, sem_i, sem_o = bufs[ci % NBUF]
        off = base + ci * ROWS_C
        pltpu.make_async_copy(in_hbm.at[pl.ds(off, ROWS_C), :], in_v, sem_i).wait()
        pltpu.make_async_copy(tg_hbm.at[pl.ds(off, ROWS_C), :], tg_v, sem_i).wait()
        if ci >= NBUF:
            prev = base + (ci - NBUF) * ROWS_C
            pltpu.make_async_copy(
                out_v, out_hbm.at[pl.ds(prev, ROWS_C), :], sem_o).wait()

        @plsc.parallel_loop(0, ROWS_C, 1, unroll=2)
        def _row(r):
            for c in WINS:
                s = (r, pl.ds(c, 16))
                idx = (in_v[s] * 4 + tg_v[s]) & 15
                out_v[s] = tab.at[idx].get(mode="promise_in_bounds")

        pltpu.async_copy(out_v, out_hbm.at[pl.ds(off, ROWS_C), :], sem_o)
        if ci + NBUF < NCHUNK:
            noff = base + (ci + NBUF) * ROWS_C
            pltpu.async_copy(in_hbm.at[pl.ds(noff, ROWS_C), :], in_v, sem_i)
            pltpu.async_copy(tg_hbm.at[pl.ds(noff, ROWS_C), :], tg_v, sem_i)

    for ci in range(NCHUNK - NBUF, NCHUNK):
        _, _, out_v, _, sem_o = bufs[ci % NBUF]
        off = base + ci * ROWS_C
        pltpu.make_async_copy(out_v, out_hbm.at[pl.ds(off, ROWS_C), :], sem_o).wait()


def kernel(input_, target, sim_lookup):
    return _sc_lookup(
        input_.astype(jnp.int32),
        target.astype(jnp.int32),
        sim_lookup.astype(jnp.float32),
    )


# speedup vs baseline: 3.4354x; 1.7940x over previous
import functools

import jax
import jax.numpy as jnp
from jax import lax
from jax.experimental import pallas as pl
from jax.experimental.pallas import tpu as pltpu
from jax.experimental.pallas import tpu_sc as plsc

R, C = 100, 16384
NC, NS = 2, 16
NW = NC * NS
COLS_W = C // NW
NBUF = 2
COLS_C = 128
NCHUNK = COLS_W // COLS_C

_mesh = plsc.VectorSubcoreMesh(
    core_axis_name="c", subcore_axis_name="s", num_cores=NC, num_subcores=NS
)


@functools.partial(
    pl.kernel,
    out_type=jax.ShapeDtypeStruct((R, C), jnp.float32),
    mesh=_mesh,
    compiler_params=pltpu.CompilerParams(use_tc_tiling_on_sc=True),
    scratch_types=[
        pltpu.VMEM((16,), jnp.float32),
        pltpu.VMEM((R, COLS_C), jnp.int32), pltpu.VMEM((R, COLS_C), jnp.int32),
        pltpu.VMEM((R, COLS_C), jnp.int32), pltpu.VMEM((R, COLS_C), jnp.int32),
        pltpu.VMEM((R, COLS_C), jnp.float32), pltpu.VMEM((R, COLS_C), jnp.float32),
        pltpu.SemaphoreType.DMA, pltpu.SemaphoreType.DMA,
        pltpu.SemaphoreType.DMA, pltpu.SemaphoreType.DMA,
    ],
)
def _sc_lookup(in_hbm, tg_hbm, tab_hbm, out_hbm,
               tab_v, in0, in1, tg0, tg1, out0, out1,
               si0, si1, so0, so1):
    wid = lax.axis_index("s") * NC + lax.axis_index("c")
    base = wid * COLS_W
    pltpu.sync_copy(tab_hbm, tab_v)
    tab = tab_v[...]

    bufs = ((in0, tg0, out0, si0, so0), (in1, tg1, out1, si1, so1))

    for b in range(NBUF):
        off = base + b * COLS_C
        in_v, tg_v, _, sem_i, _ = bufs[b]
        pltpu.async_copy(in_hbm.at[:, pl.ds(off, COLS_C)], in_v, sem_i)
        pltpu.async_copy(tg_hbm.at[:, pl.ds(off, COLS_C)], tg_v, sem_i)

    for ci in range(NCHUNK):
        in_v, tg_v, out_v, sem_i, sem_o = bufs[ci % NBUF]
        off = base + ci * COLS_C
        pltpu.make_async_copy(in_hbm.at[:, pl.ds(off, COLS_C)], in_v, sem_i).wait()
        pltpu.make_async_copy(tg_hbm.at[:, pl.ds(off, COLS_C)], tg_v, sem_i).wait()
        if ci >= NBUF:
            prev = base + (ci - NBUF) * COLS_C
            pltpu.make_async_copy(
                out_v, out_hbm.at[:, pl.ds(prev, COLS_C)], sem_o).wait()

        @plsc.parallel_loop(0, R, 1, unroll=2)
        def _row(r):
            for c in range(0, COLS_C, 16):
                s = (r, pl.ds(c, 16))
                idx = (in_v[s] * 4 + tg_v[s]) & 15
                out_v[s] = tab.at[idx].get(mode="promise_in_bounds")

        pltpu.async_copy(out_v, out_hbm.at[:, pl.ds(off, COLS_C)], sem_o)
        if ci + NBUF < NCHUNK:
            noff = base + (ci + NBUF) * COLS_C
            pltpu.async_copy(in_hbm.at[:, pl.ds(noff, COLS_C)], in_v, sem_i)
            pltpu.async_copy(tg_hbm.at[:, pl.ds(noff, COLS_C)], tg_v, sem_i)

    for ci in range(NCHUNK - NBUF, NCHUNK):
        _, _, out_v, _, sem_o = bufs[ci % NBUF]
        off = base + ci * COLS_C
        pltpu.make_async_copy(out_v, out_hbm.at[:, pl.ds(off, COLS_C)], sem_o).wait()


def kernel(input_, target, sim_lookup):
    out_t = _sc_lookup(
        input_.T.astype(jnp.int32),
        target.T.astype(jnp.int32),
        sim_lookup.astype(jnp.float32),
    )
    return out_t.T
